# Initial kernel scaffold; baseline (speedup 1.0000x reference)
import jax
import jax.numpy as jnp
from jax.experimental import pallas as pl


def kernel(input_word_ids, input_type_ids, word_emb, pos_emb, type_emb, ln_gamma, ln_beta):
    raise NotImplementedError("write your pallas kernel here")


# speedup vs baseline: 1.5530x; 1.5530x over previous
import functools

import jax
import jax.numpy as jnp
from jax import lax
from jax.experimental import pallas as pl
from jax.experimental.pallas import tpu as pltpu
from jax.experimental.pallas import tpu_sc as plsc

_H = 1024
_NW = 32
_CHUNK = 64
_LN_EPS = 1e-3


def _sc_gather_rows(table, ids_flat):
    ntok = ids_flat.shape[0]
    tok_per_w = ntok // _NW
    nchunk = tok_per_w // _CHUNK
    mesh = plsc.VectorSubcoreMesh(core_axis_name="c", subcore_axis_name="s")

    @functools.partial(
        pl.kernel,
        out_type=jax.ShapeDtypeStruct((ntok, _H), jnp.float32),
        mesh=mesh,
        scratch_types=[
            pltpu.VMEM((nchunk, _CHUNK), jnp.int32),
            pltpu.VMEM((_CHUNK, _H), jnp.float32),
            pltpu.SemaphoreType.DMA,
        ],
    )
    def gather_kernel(table_hbm, idx_hbm, out_hbm, idx_v, rows_v, sem):
        wid = lax.axis_index("s") * 2 + lax.axis_index("c")
        base = wid * tok_per_w
        for c in range(nchunk):
            pltpu.sync_copy(idx_hbm.at[pl.ds(base + c * _CHUNK, _CHUNK)],
                            idx_v.at[c])
        for c in range(nchunk):
            pltpu.async_copy(table_hbm.at[idx_v.at[c]], rows_v, sem).wait()
            pltpu.sync_copy(rows_v,
                            out_hbm.at[pl.ds(base + c * _CHUNK, _CHUNK)])

    return gather_kernel(table, ids_flat)


def _ln_body(g_ref, pos_ref, tid_ref, type_ref, gamma_ref, beta_ref, o_ref):
    t0 = type_ref[0:1, :]
    t1 = type_ref[1:2, :]
    is_one = tid_ref[0] > 0.5
    type_row = jnp.where(is_one, t1, t0)
    x = g_ref[...] + pos_ref[...] + type_row
    mu = jnp.mean(x, axis=1, keepdims=True)
    xc = x - mu
    var = jnp.mean(xc * xc, axis=1, keepdims=True)
    y = xc * lax.rsqrt(var + _LN_EPS)
    o_ref[...] = y * gamma_ref[...] + beta_ref[...]


def _tc_add_ln(gathered, type_f, pos_emb, type_emb, gamma, beta,
               batch, seq, tok_blk, interpret=False):
    sblk = seq // tok_blk
    grid = (sblk, batch)

    return pl.pallas_call(
        _ln_body,
        grid=grid,
        in_specs=[
            pl.BlockSpec((tok_blk, _H), lambda s, b: (b * sblk + s, 0)),
            pl.BlockSpec((tok_blk, _H), lambda s, b: (s, 0)),
            pl.BlockSpec((1, tok_blk, 1), lambda s, b: (b * sblk + s, 0, 0)),
            pl.BlockSpec((2, _H), lambda s, b: (0, 0)),
            pl.BlockSpec((1, _H), lambda s, b: (0, 0)),
            pl.BlockSpec((1, _H), lambda s, b: (0, 0)),
        ],
        out_specs=pl.BlockSpec((tok_blk, _H), lambda s, b: (b * sblk + s, 0)),
        out_shape=jax.ShapeDtypeStruct((batch * seq, _H), jnp.float32),
        interpret=interpret,
    )(gathered, pos_emb, type_f, type_emb, gamma, beta)


def kernel(input_word_ids, input_type_ids, word_emb, pos_emb, type_emb,
           ln_gamma, ln_beta):
    batch, seq = input_word_ids.shape
    ntok = batch * seq
    tok_blk = ntok // _NW

    ids_flat = input_word_ids.reshape(ntok).astype(jnp.int32)
    type_f = input_type_ids.astype(jnp.float32).reshape(_NW, tok_blk, 1)

    gathered = _sc_gather_rows(word_emb, ids_flat)
    out = _tc_add_ln(gathered, type_f, pos_emb, type_emb,
                     ln_gamma.reshape(1, _H), ln_beta.reshape(1, _H),
                     batch, seq, tok_blk)
    return out.reshape(batch, seq, _H)


# speedup vs baseline: 1.7267x; 1.1118x over previous
import functools

import jax
import jax.numpy as jnp
from jax import lax
from jax.experimental import pallas as pl
from jax.experimental.pallas import tpu as pltpu
from jax.experimental.pallas import tpu_sc as plsc

_H = 1024
_NW = 32
_CHUNK = 32
_LN_EPS = 1e-3


def _sc_gather_rows(table, ids_flat):
    ntok = ids_flat.shape[0]
    tok_per_w = ntok // _NW
    nchunk = tok_per_w // _CHUNK
    mesh = plsc.VectorSubcoreMesh(core_axis_name="c", subcore_axis_name="s")

    @functools.partial(
        pl.kernel,
        out_type=jax.ShapeDtypeStruct((ntok, _H), jnp.float32),
        mesh=mesh,
        scratch_types=[
            pltpu.VMEM((nchunk, _CHUNK), jnp.int32),
            pltpu.VMEM((2, _CHUNK, _H), jnp.float32),
            pltpu.SemaphoreType.DMA,
            pltpu.SemaphoreType.DMA,
            pltpu.SemaphoreType.DMA,
            pltpu.SemaphoreType.DMA,
        ],
    )
    def gather_kernel(table_hbm, idx_hbm, out_hbm, idx_v, bufs,
                      gsem0, gsem1, osem0, osem1):
        wid = lax.axis_index("s") * 2 + lax.axis_index("c")
        base = wid * tok_per_w
        gsems = (gsem0, gsem1)
        osems = (osem0, osem1)
        for c in range(nchunk):
            pltpu.sync_copy(idx_hbm.at[pl.ds(base + c * _CHUNK, _CHUNK)],
                            idx_v.at[c])
        ocopy = [None, None]
        gcopy = [None, None]
        for c in range(nchunk):
            b = c % 2
            if ocopy[b] is not None:
                ocopy[b].wait()
            gcopy[b] = pltpu.async_copy(table_hbm.at[idx_v.at[c]],
                                        bufs.at[b], gsems[b])
            if c > 0:
                pb = (c - 1) % 2
                gcopy[pb].wait()
                ocopy[pb] = pltpu.async_copy(
                    bufs.at[pb],
                    out_hbm.at[pl.ds(base + (c - 1) * _CHUNK, _CHUNK)],
                    osems[pb])
        lb = (nchunk - 1) % 2
        gcopy[lb].wait()
        ocopy[lb] = pltpu.async_copy(
            bufs.at[lb],
            out_hbm.at[pl.ds(base + (nchunk - 1) * _CHUNK, _CHUNK)],
            osems[lb])
        for b in range(2):
            if ocopy[b] is not None:
                ocopy[b].wait()

    return gather_kernel(table, ids_flat)


def _ln_body(g_ref, pos_ref, tid_ref, type_ref, gamma_ref, beta_ref, o_ref):
    t0 = type_ref[0:1, :]
    t1 = type_ref[1:2, :]
    is_one = tid_ref[0] > 0.5
    type_row = jnp.where(is_one, t1, t0)
    x = g_ref[...] + pos_ref[...] + type_row
    mu = jnp.mean(x, axis=1, keepdims=True)
    xc = x - mu
    var = jnp.mean(xc * xc, axis=1, keepdims=True)
    y = xc * lax.rsqrt(var + _LN_EPS)
    o_ref[...] = y * gamma_ref[...] + beta_ref[...]


def _tc_add_ln(gathered, type_f, pos_emb, type_emb, gamma, beta,
               batch, seq, tok_blk, interpret=False):
    sblk = seq // tok_blk
    grid = (sblk, batch)

    return pl.pallas_call(
        _ln_body,
        grid=grid,
        in_specs=[
            pl.BlockSpec((tok_blk, _H), lambda s, b: (b * sblk + s, 0)),
            pl.BlockSpec((tok_blk, _H), lambda s, b: (s, 0)),
            pl.BlockSpec((1, tok_blk, 1), lambda s, b: (b * sblk + s, 0, 0)),
            pl.BlockSpec((2, _H), lambda s, b: (0, 0)),
            pl.BlockSpec((1, _H), lambda s, b: (0, 0)),
            pl.BlockSpec((1, _H), lambda s, b: (0, 0)),
        ],
        out_specs=pl.BlockSpec((tok_blk, _H), lambda s, b: (b * sblk + s, 0)),
        out_shape=jax.ShapeDtypeStruct((batch * seq, _H), jnp.float32),
        interpret=interpret,
    )(gathered, pos_emb, type_f, type_emb, gamma, beta)


def kernel(input_word_ids, input_type_ids, word_emb, pos_emb, type_emb,
           ln_gamma, ln_beta):
    batch, seq = input_word_ids.shape
    ntok = batch * seq
    tok_blk = 512

    ids_flat = input_word_ids.reshape(ntok).astype(jnp.int32)
    type_f = input_type_ids.astype(jnp.float32).reshape(ntok // tok_blk,
                                                        tok_blk, 1)

    gathered = _sc_gather_rows(word_emb, ids_flat)
    out = _tc_add_ln(gathered, type_f, pos_emb, type_emb,
                     ln_gamma.reshape(1, _H), ln_beta.reshape(1, _H),
                     batch, seq, tok_blk)
    return out.reshape(batch, seq, _H)
